# baseline (speedup 1.0000x reference)
import functools

import jax
import jax.numpy as jnp
from jax import lax
from jax.experimental import pallas as pl
from jax.experimental.pallas import tpu as pltpu
from jax.experimental.pallas import tpu_sc as plsc

N = 10000
DH = 256
NC = 2
NS = 16
L = 16
EB = 128
TR = 624
TAIL0 = NS * TR
TAIL = N - TAIL0
DUMMY = N
ACC_ROWS = N + 16
F32 = jnp.float32


def _sc_mesh():
    return plsc.VectorSubcoreMesh(core_axis_name="c", subcore_axis_name="s")


@functools.lru_cache(maxsize=None)
def _make_deg(nblk):
    half = nblk // 2

    @functools.partial(
        pl.kernel,
        out_type=jax.ShapeDtypeStruct((2 * N, 128), F32),
        mesh=_sc_mesh(),
        scratch_types=[
            pltpu.VMEM((nblk, EB), jnp.int32),
            pltpu.VMEM((EB, 128), F32),
            pltpu.VMEM_SHARED((ACC_ROWS, 128), F32),
        ],
    )
    def deg_kernel(dst_hbm, ones_hbm, init_hbm, out_hbm, dst_v, ones_v, acc):
        c = lax.axis_index("c")
        s = lax.axis_index("s")
        base = s * TR
        pltpu.sync_copy(dst_hbm.at[s], dst_v)
        pltpu.sync_copy(ones_hbm, ones_v)
        pltpu.sync_copy(init_hbm.at[c, pl.ds(0, TR)], acc.at[pl.ds(base, TR)])

        @pl.when(s == 0)
        def _():
            pltpu.sync_copy(init_hbm.at[c, pl.ds(0, ACC_ROWS - TAIL0)],
                            acc.at[pl.ds(TAIL0, ACC_ROWS - TAIL0)])

        plsc.subcore_barrier()

        def blk(j, carry):
            pltpu.sync_copy(ones_v, acc.at[dst_v.at[j]], add=True)
            return carry

        lax.fori_loop(c * half, (c + 1) * half, blk, 0)
        plsc.subcore_barrier()
        pltpu.sync_copy(acc.at[pl.ds(base, TR)],
                        out_hbm.at[pl.ds(c * N + base, TR)])

        @pl.when(s == 0)
        def _():
            pltpu.sync_copy(acc.at[pl.ds(TAIL0, TAIL)],
                            out_hbm.at[pl.ds(c * N + TAIL0, TAIL)])

    return deg_kernel


@functools.lru_cache(maxsize=None)
def _make_prop(n_chunks, nblk):
    rows = n_chunks * N

    SB = 2 * EB
    nsb = nblk // 2

    @functools.partial(
        pl.kernel,
        out_type=jax.ShapeDtypeStruct((rows, 128), F32),
        mesh=_sc_mesh(),
        scratch_types=[
            pltpu.VMEM((SB,), jnp.int32),
            pltpu.VMEM((SB,), jnp.int32),
            pltpu.VMEM((SB,), jnp.int32),
            pltpu.VMEM((SB,), jnp.int32),
            pltpu.VMEM((SB, 128), F32),
            pltpu.VMEM_SHARED((ACC_ROWS, 128), F32),
            pltpu.SemaphoreType.DMA,
            pltpu.SemaphoreType.DMA,
            pltpu.SemaphoreType.DMA,
            pltpu.SemaphoreType.DMA,
        ],
    )
    def prop_kernel(srck_hbm, dst_hbm, y_hbm, out_hbm,
                    iba, ibb, dba, dbb, mb, acc,
                    isema, isemb, dsema, dsemb):
        c = lax.axis_index("c")
        s = lax.axis_index("s")
        base = s * TR
        last = nsb - 1
        for i in range(n_chunks // 2):
            k = 2 * i + c
            row0 = k * N
            pltpu.sync_copy(y_hbm.at[pl.ds(row0 + base, TR)],
                            acc.at[pl.ds(base, TR)])

            @pl.when(s == 0)
            def _():
                pltpu.sync_copy(y_hbm.at[pl.ds(row0 + TAIL0, TAIL)],
                                acc.at[pl.ds(TAIL0, TAIL)])

            plsc.subcore_barrier()

            pltpu.async_copy(srck_hbm.at[k, s, pl.ds(0, SB)], iba, isema)
            pltpu.async_copy(dst_hbm.at[s, pl.ds(0, SB)], dba, dsema)
            pltpu.async_copy(srck_hbm.at[k, s, pl.ds(SB, SB)], ibb, isemb)
            pltpu.async_copy(dst_hbm.at[s, pl.ds(SB, SB)], dbb, dsemb)

            def blk(t, carry):
                g0 = 2 * t
                g1 = 2 * t + 1
                n0 = SB * lax.min(g0 + 2, last)
                n1 = SB * lax.min(g1 + 2, last)
                pltpu.make_async_copy(srck_hbm.at[k, s, pl.ds(0, SB)], iba,
                                      isema).wait()
                pltpu.sync_copy(y_hbm.at[iba], mb)
                pltpu.async_copy(srck_hbm.at[k, s, pl.ds(n0, SB)],
                                 iba, isema)
                pltpu.make_async_copy(dst_hbm.at[s, pl.ds(0, SB)], dba,
                                      dsema).wait()
                pltpu.sync_copy(mb, acc.at[dba], add=True)
                pltpu.async_copy(dst_hbm.at[s, pl.ds(n0, SB)], dba, dsema)
                pltpu.make_async_copy(srck_hbm.at[k, s, pl.ds(0, SB)], ibb,
                                      isemb).wait()
                pltpu.sync_copy(y_hbm.at[ibb], mb)
                pltpu.async_copy(srck_hbm.at[k, s, pl.ds(n1, SB)],
                                 ibb, isemb)
                pltpu.make_async_copy(dst_hbm.at[s, pl.ds(0, SB)], dbb,
                                      dsemb).wait()
                pltpu.sync_copy(mb, acc.at[dbb], add=True)
                pltpu.async_copy(dst_hbm.at[s, pl.ds(n1, SB)], dbb, dsemb)
                return carry

            lax.fori_loop(0, nsb // 2, blk, 0)
            pltpu.make_async_copy(srck_hbm.at[k, s, pl.ds(0, SB)], iba,
                                  isema).wait()
            pltpu.make_async_copy(srck_hbm.at[k, s, pl.ds(0, SB)], ibb,
                                  isemb).wait()
            pltpu.make_async_copy(dst_hbm.at[s, pl.ds(0, SB)], dba,
                                  dsema).wait()
            pltpu.make_async_copy(dst_hbm.at[s, pl.ds(0, SB)], dbb,
                                  dsemb).wait()
            plsc.subcore_barrier()
            pltpu.sync_copy(acc.at[pl.ds(base, TR)],
                            out_hbm.at[pl.ds(row0 + base, TR)])

            @pl.when(s == 0)
            def _():
                pltpu.sync_copy(acc.at[pl.ds(TAIL0, TAIL)],
                                out_hbm.at[pl.ds(row0 + TAIL0, TAIL)])

            plsc.subcore_barrier()

    return prop_kernel


RB = 2000


def _scale_body(degp1_ref, x_ref, h1_ref, h2_ref, dinv_ref, y_ref):
    dinv = lax.rsqrt(degp1_ref[0, :, :16] + degp1_ref[1, :, :16])
    dinv_ref[...] = dinv
    dv = dinv[:, 0:1]
    y_ref[0] = x_ref[:, :128] * dv
    y_ref[1] = x_ref[:, 128:] * dv
    y_ref[2] = h1_ref[:, :128] * dv
    y_ref[3] = h1_ref[:, 128:] * dv
    y_ref[4] = h2_ref[:, :128] * dv
    y_ref[5] = h2_ref[:, 128:] * dv


def _scale_call(degp1, x, h1, h2):
    return pl.pallas_call(
        _scale_body,
        grid=(N // RB,),
        in_specs=[
            pl.BlockSpec((2, RB, 128), lambda i: (0, i, 0)),
            pl.BlockSpec((RB, DH), lambda i: (i, 0)),
            pl.BlockSpec((RB, DH), lambda i: (i, 0)),
            pl.BlockSpec((RB, DH), lambda i: (i, 0)),
        ],
        out_specs=[
            pl.BlockSpec((RB, 16), lambda i: (i, 0)),
            pl.BlockSpec((6, RB, 128), lambda i: (0, i, 0)),
        ],
        out_shape=[
            jax.ShapeDtypeStruct((N, 16), F32),
            jax.ShapeDtypeStruct((6, N, 128), F32),
        ],
    )(degp1, x, h1, h2)


def _zr_body(dx_ref, dh_ref, dinv_ref, h_ref, wzr_ref, bzr_ref, wxh_ref,
             z_ref, yrh_ref, xh_ref):
    dv = dinv_ref[:, 0:1]
    u0 = dx_ref[0] * dv
    u1 = dx_ref[1] * dv
    v0 = dh_ref[0] * dv
    v1 = dh_ref[1] * dv
    zr = (jnp.dot(u0, wzr_ref[0], preferred_element_type=F32)
          + jnp.dot(u1, wzr_ref[1], preferred_element_type=F32)
          + jnp.dot(v0, wzr_ref[2], preferred_element_type=F32)
          + jnp.dot(v1, wzr_ref[3], preferred_element_type=F32)
          + bzr_ref[...])
    z = jax.nn.sigmoid(zr[:, :DH])
    r = jax.nn.sigmoid(zr[:, DH:])
    z_ref[...] = z
    rh = r * h_ref[...]
    yrh_ref[0] = rh[:, :128] * dv
    yrh_ref[1] = rh[:, 128:] * dv
    xh_ref[...] = (jnp.dot(u0, wxh_ref[0], preferred_element_type=F32)
                   + jnp.dot(u1, wxh_ref[1], preferred_element_type=F32))


def _zr_call(dx, dh, dinv16, hl, wzr, bzr, wxh):
    return pl.pallas_call(
        _zr_body,
        grid=(N // RB,),
        in_specs=[
            pl.BlockSpec((2, RB, 128), lambda i: (0, i, 0)),
            pl.BlockSpec((2, RB, 128), lambda i: (0, i, 0)),
            pl.BlockSpec((RB, 16), lambda i: (i, 0)),
            pl.BlockSpec((RB, DH), lambda i: (i, 0)),
            pl.BlockSpec((4, 128, 2 * DH), lambda i: (0, 0, 0)),
            pl.BlockSpec((1, 2 * DH), lambda i: (0, 0)),
            pl.BlockSpec((2, 128, DH), lambda i: (0, 0, 0)),
        ],
        out_specs=[
            pl.BlockSpec((RB, DH), lambda i: (i, 0)),
            pl.BlockSpec((2, RB, 128), lambda i: (0, i, 0)),
            pl.BlockSpec((RB, DH), lambda i: (i, 0)),
        ],
        out_shape=[
            jax.ShapeDtypeStruct((N, DH), F32),
            jax.ShapeDtypeStruct((2, N, 128), F32),
            jax.ShapeDtypeStruct((N, DH), F32),
        ],
    )(dx, dh, dinv16, hl, wzr, bzr, wxh)


def _h_body(drh_ref, dinv_ref, xh_ref, z_ref, h_ref, whh_ref, bh_ref,
            hp_ref, yx_ref):
    dv = dinv_ref[:, 0:1]
    w0 = drh_ref[0] * dv
    w1 = drh_ref[1] * dv
    ht = jnp.tanh(xh_ref[...]
                  + jnp.dot(w0, whh_ref[0], preferred_element_type=F32)
                  + jnp.dot(w1, whh_ref[1], preferred_element_type=F32)
                  + bh_ref[...])
    z = z_ref[...]
    hp = z * h_ref[...] + (1.0 - z) * ht
    hp_ref[...] = hp
    yx_ref[0] = hp[:, :128] * dv
    yx_ref[1] = hp[:, 128:] * dv


def _h_call(drh, dinv16, xh, z, hl, whh, bh):
    return pl.pallas_call(
        _h_body,
        grid=(N // RB,),
        in_specs=[
            pl.BlockSpec((2, RB, 128), lambda i: (0, i, 0)),
            pl.BlockSpec((RB, 16), lambda i: (i, 0)),
            pl.BlockSpec((RB, DH), lambda i: (i, 0)),
            pl.BlockSpec((RB, DH), lambda i: (i, 0)),
            pl.BlockSpec((RB, DH), lambda i: (i, 0)),
            pl.BlockSpec((2, 128, DH), lambda i: (0, 0, 0)),
            pl.BlockSpec((1, DH), lambda i: (0, 0)),
        ],
        out_specs=[
            pl.BlockSpec((RB, DH), lambda i: (i, 0)),
            pl.BlockSpec((2, RB, 128), lambda i: (0, i, 0)),
        ],
        out_shape=[
            jax.ShapeDtypeStruct((N, DH), F32),
            jax.ShapeDtypeStruct((2, N, 128), F32),
        ],
    )(drh, dinv16, xh, z, hl, whh, bh)


def _layer_weights(p):
    top = jnp.concatenate([p["Wxz"], p["Wxr"]], axis=1)
    bot = jnp.concatenate([p["Whz"], p["Whr"]], axis=1)
    wzr = jnp.concatenate([top, bot], axis=0).reshape(4, 128, 2 * DH)
    bzr = jnp.concatenate([p["bxz"] + p["bhz"],
                           p["bxr"] + p["bhr"]]).reshape(1, 2 * DH)
    wxh = p["Wxh"].reshape(2, 128, DH)
    whh = p["Whh"].reshape(2, 128, DH)
    bh = (p["bxh"] + p["bhh"]).reshape(1, DH)
    return wzr, bzr, wxh, whh, bh


def kernel(inp, edgidx, h, params):
    src = edgidx[0].astype(jnp.int32)
    dst = edgidx[1].astype(jnp.int32)
    e = src.shape[0]
    nblk = 4 * (-(-e // (4 * NS * EB)))
    pad = NS * nblk * EB - e
    src3 = jnp.concatenate(
        [src, jnp.zeros((pad,), jnp.int32)]).reshape(NS, nblk, EB)
    dst3 = jnp.concatenate(
        [dst, jnp.full((pad,), DUMMY, jnp.int32)]).reshape(NS, nblk, EB)
    srcf = src3.reshape(NS, nblk * EB)
    offs = (jnp.arange(6, dtype=jnp.int32) * N)[:, None, None]
    src6 = srcf[None] + offs
    src2 = src6[:2]
    dstf = dst3.reshape(NS, nblk * EB)

    prop6 = _make_prop(6, nblk)
    prop2 = _make_prop(2, nblk)

    ones = jnp.ones((EB, 128), F32)
    init = jnp.stack([jnp.ones((TR, 128), F32), jnp.zeros((TR, 128), F32)])
    degp1 = _make_deg(nblk)(dst3, ones, init).reshape(2, N, 128)

    h1, h2 = h[0], h[1]
    dinv16, y6 = _scale_call(degp1, inp, h1, h2)

    d6 = prop6(src6, dstf, y6.reshape(6 * N, 128)).reshape(6, N, 128)
    dx1, dh1, dh2 = d6[0:2], d6[2:4], d6[4:6]

    wzr1, bzr1, wxh1, whh1, bh1 = _layer_weights(params[0])
    wzr2, bzr2, wxh2, whh2, bh2 = _layer_weights(params[1])

    z1, yrh1, xh1 = _zr_call(dx1, dh1, dinv16, h1, wzr1, bzr1, wxh1)
    drh1 = prop2(src2, dstf, yrh1.reshape(2 * N, 128)).reshape(2, N, 128)
    hp1, yx2 = _h_call(drh1, dinv16, xh1, z1, h1, whh1, bh1)

    dx2 = prop2(src2, dstf, yx2.reshape(2 * N, 128)).reshape(2, N, 128)
    z2, yrh2, xh2 = _zr_call(dx2, dh2, dinv16, h2, wzr2, bzr2, wxh2)
    drh2 = prop2(src2, dstf, yrh2.reshape(2 * N, 128)).reshape(2, N, 128)
    hp2, _ = _h_call(drh2, dinv16, xh2, z2, h2, whh2, bh2)

    h_out = jnp.stack([hp1, hp2], axis=0)
    return (h_out, h_out)


# speedup vs baseline: 1.2815x; 1.2815x over previous
import functools

import jax
import jax.numpy as jnp
from jax import lax
from jax.experimental import pallas as pl
from jax.experimental.pallas import tpu as pltpu
from jax.experimental.pallas import tpu_sc as plsc

N = 10000
DH = 256
NC = 2
NS = 16
L = 16
EB = 128
EBP = 104
TR = 624
TAIL0 = NS * TR
TAIL = N - TAIL0
DUMMY = N
ACC_ROWS = N + 16
F32 = jnp.float32


def _sc_mesh():
    return plsc.VectorSubcoreMesh(core_axis_name="c", subcore_axis_name="s")


@functools.lru_cache(maxsize=None)
def _make_deg(nblk):
    half = nblk // 2

    @functools.partial(
        pl.kernel,
        out_type=jax.ShapeDtypeStruct((2 * N, 128), F32),
        mesh=_sc_mesh(),
        scratch_types=[
            pltpu.VMEM((nblk, EB), jnp.int32),
            pltpu.VMEM((EB, 128), F32),
            pltpu.VMEM_SHARED((ACC_ROWS, 128), F32),
        ],
    )
    def deg_kernel(dst_hbm, ones_hbm, init_hbm, out_hbm, dst_v, ones_v, acc):
        c = lax.axis_index("c")
        s = lax.axis_index("s")
        base = s * TR
        pltpu.sync_copy(dst_hbm.at[s], dst_v)
        pltpu.sync_copy(ones_hbm, ones_v)
        pltpu.sync_copy(init_hbm.at[c, pl.ds(0, TR)], acc.at[pl.ds(base, TR)])

        @pl.when(s == 0)
        def _():
            pltpu.sync_copy(init_hbm.at[c, pl.ds(0, ACC_ROWS - TAIL0)],
                            acc.at[pl.ds(TAIL0, ACC_ROWS - TAIL0)])

        plsc.subcore_barrier()

        def blk(j, carry):
            pltpu.sync_copy(ones_v, acc.at[dst_v.at[j]], add=True)
            return carry

        lax.fori_loop(c * half, (c + 1) * half, blk, 0)
        plsc.subcore_barrier()
        pltpu.sync_copy(acc.at[pl.ds(base, TR)],
                        out_hbm.at[pl.ds(c * N + base, TR)])

        @pl.when(s == 0)
        def _():
            pltpu.sync_copy(acc.at[pl.ds(TAIL0, TAIL)],
                            out_hbm.at[pl.ds(c * N + TAIL0, TAIL)])

    return deg_kernel


@functools.lru_cache(maxsize=None)
def _make_prop(n_chunks, nblk):
    rows = n_chunks * N
    ept = nblk * EBP

    @functools.partial(
        pl.kernel,
        out_type=jax.ShapeDtypeStruct((rows, 128), F32),
        mesh=_sc_mesh(),
        scratch_types=[
            pltpu.VMEM((ept,), jnp.int32),
            pltpu.VMEM((nblk, EBP), jnp.int32),
            pltpu.VMEM((EBP, 128), F32),
            pltpu.VMEM((EBP, 128), F32),
            pltpu.VMEM_SHARED((ACC_ROWS, 128), F32),
            pltpu.SemaphoreType.DMA,
            pltpu.SemaphoreType.DMA,
        ],
    )
    def prop_kernel(src_hbm, dst_hbm, y_hbm, out_hbm,
                    src_v, dst_v, mb0, mb1, acc, sem0, sem1):
        c = lax.axis_index("c")
        s = lax.axis_index("s")
        base = s * TR
        last = nblk - 1
        pltpu.sync_copy(src_hbm.at[s], src_v)
        pltpu.sync_copy(dst_hbm.at[s], dst_v)
        for i in range(n_chunks // 2):
            delta = c * N if i == 0 else 2 * N

            def offs(t, carry):
                sl = pl.ds(t * L, L)
                src_v[sl] = src_v[sl] + delta
                return carry

            lax.fori_loop(0, ept // L, offs, 0)
            k = 2 * i + c
            row0 = k * N
            pltpu.sync_copy(y_hbm.at[pl.ds(row0 + base, TR)],
                            acc.at[pl.ds(base, TR)])

            @pl.when(s == 0)
            def _():
                pltpu.sync_copy(y_hbm.at[pl.ds(row0 + TAIL0, TAIL)],
                                acc.at[pl.ds(TAIL0, TAIL)])

            plsc.subcore_barrier()

            pltpu.async_copy(y_hbm.at[src_v.at[pl.ds(0, EBP)]], mb0, sem0)

            def blk(g, carry):
                j0 = 2 * g
                j1 = 2 * g + 1
                pltpu.async_copy(y_hbm.at[src_v.at[pl.ds(j1 * EBP, EBP)]],
                                 mb1, sem1)
                pltpu.make_async_copy(y_hbm.at[pl.ds(0, EBP)], mb0,
                                      sem0).wait()
                pltpu.sync_copy(mb0, acc.at[dst_v.at[j0]], add=True)
                jn = lax.min(j0 + 2, last)
                pltpu.async_copy(y_hbm.at[src_v.at[pl.ds(jn * EBP, EBP)]],
                                 mb0, sem0)
                pltpu.make_async_copy(y_hbm.at[pl.ds(0, EBP)], mb1,
                                      sem1).wait()
                pltpu.sync_copy(mb1, acc.at[dst_v.at[j1]], add=True)
                return carry

            lax.fori_loop(0, nblk // 2, blk, 0)
            pltpu.make_async_copy(y_hbm.at[pl.ds(0, EBP)], mb0, sem0).wait()
            plsc.subcore_barrier()
            pltpu.sync_copy(acc.at[pl.ds(base, TR)],
                            out_hbm.at[pl.ds(row0 + base, TR)])

            @pl.when(s == 0)
            def _():
                pltpu.sync_copy(acc.at[pl.ds(TAIL0, TAIL)],
                                out_hbm.at[pl.ds(row0 + TAIL0, TAIL)])

            plsc.subcore_barrier()

    return prop_kernel


RB = 2000


def _scale_body(degp1_ref, x_ref, h1_ref, h2_ref, dinv_ref, y_ref):
    dinv = lax.rsqrt(degp1_ref[0, :, :16] + degp1_ref[1, :, :16])
    dinv_ref[...] = dinv
    dv = dinv[:, 0:1]
    y_ref[0] = x_ref[:, :128] * dv
    y_ref[1] = x_ref[:, 128:] * dv
    y_ref[2] = h1_ref[:, :128] * dv
    y_ref[3] = h1_ref[:, 128:] * dv
    y_ref[4] = h2_ref[:, :128] * dv
    y_ref[5] = h2_ref[:, 128:] * dv


def _scale_call(degp1, x, h1, h2):
    return pl.pallas_call(
        _scale_body,
        grid=(N // RB,),
        in_specs=[
            pl.BlockSpec((2, RB, 128), lambda i: (0, i, 0)),
            pl.BlockSpec((RB, DH), lambda i: (i, 0)),
            pl.BlockSpec((RB, DH), lambda i: (i, 0)),
            pl.BlockSpec((RB, DH), lambda i: (i, 0)),
        ],
        out_specs=[
            pl.BlockSpec((RB, 16), lambda i: (i, 0)),
            pl.BlockSpec((6, RB, 128), lambda i: (0, i, 0)),
        ],
        out_shape=[
            jax.ShapeDtypeStruct((N, 16), F32),
            jax.ShapeDtypeStruct((6, N, 128), F32),
        ],
    )(degp1, x, h1, h2)


def _zr_body(dx_ref, dh_ref, dinv_ref, h_ref, wzr_ref, bzr_ref, wxh_ref,
             z_ref, yrh_ref, xh_ref):
    dv = dinv_ref[:, 0:1]
    u0 = dx_ref[0] * dv
    u1 = dx_ref[1] * dv
    v0 = dh_ref[0] * dv
    v1 = dh_ref[1] * dv
    zr = (jnp.dot(u0, wzr_ref[0], preferred_element_type=F32)
          + jnp.dot(u1, wzr_ref[1], preferred_element_type=F32)
          + jnp.dot(v0, wzr_ref[2], preferred_element_type=F32)
          + jnp.dot(v1, wzr_ref[3], preferred_element_type=F32)
          + bzr_ref[...])
    z = jax.nn.sigmoid(zr[:, :DH])
    r = jax.nn.sigmoid(zr[:, DH:])
    z_ref[...] = z
    rh = r * h_ref[...]
    yrh_ref[0] = rh[:, :128] * dv
    yrh_ref[1] = rh[:, 128:] * dv
    xh_ref[...] = (jnp.dot(u0, wxh_ref[0], preferred_element_type=F32)
                   + jnp.dot(u1, wxh_ref[1], preferred_element_type=F32))


def _zr_call(dx, dh, dinv16, hl, wzr, bzr, wxh):
    return pl.pallas_call(
        _zr_body,
        grid=(N // RB,),
        in_specs=[
            pl.BlockSpec((2, RB, 128), lambda i: (0, i, 0)),
            pl.BlockSpec((2, RB, 128), lambda i: (0, i, 0)),
            pl.BlockSpec((RB, 16), lambda i: (i, 0)),
            pl.BlockSpec((RB, DH), lambda i: (i, 0)),
            pl.BlockSpec((4, 128, 2 * DH), lambda i: (0, 0, 0)),
            pl.BlockSpec((1, 2 * DH), lambda i: (0, 0)),
            pl.BlockSpec((2, 128, DH), lambda i: (0, 0, 0)),
        ],
        out_specs=[
            pl.BlockSpec((RB, DH), lambda i: (i, 0)),
            pl.BlockSpec((2, RB, 128), lambda i: (0, i, 0)),
            pl.BlockSpec((RB, DH), lambda i: (i, 0)),
        ],
        out_shape=[
            jax.ShapeDtypeStruct((N, DH), F32),
            jax.ShapeDtypeStruct((2, N, 128), F32),
            jax.ShapeDtypeStruct((N, DH), F32),
        ],
    )(dx, dh, dinv16, hl, wzr, bzr, wxh)


def _h_body(drh_ref, dinv_ref, xh_ref, z_ref, h_ref, whh_ref, bh_ref,
            hp_ref, yx_ref):
    dv = dinv_ref[:, 0:1]
    w0 = drh_ref[0] * dv
    w1 = drh_ref[1] * dv
    ht = jnp.tanh(xh_ref[...]
                  + jnp.dot(w0, whh_ref[0], preferred_element_type=F32)
                  + jnp.dot(w1, whh_ref[1], preferred_element_type=F32)
                  + bh_ref[...])
    z = z_ref[...]
    hp = z * h_ref[...] + (1.0 - z) * ht
    hp_ref[...] = hp
    yx_ref[0] = hp[:, :128] * dv
    yx_ref[1] = hp[:, 128:] * dv


def _h_call(drh, dinv16, xh, z, hl, whh, bh):
    return pl.pallas_call(
        _h_body,
        grid=(N // RB,),
        in_specs=[
            pl.BlockSpec((2, RB, 128), lambda i: (0, i, 0)),
            pl.BlockSpec((RB, 16), lambda i: (i, 0)),
            pl.BlockSpec((RB, DH), lambda i: (i, 0)),
            pl.BlockSpec((RB, DH), lambda i: (i, 0)),
            pl.BlockSpec((RB, DH), lambda i: (i, 0)),
            pl.BlockSpec((2, 128, DH), lambda i: (0, 0, 0)),
            pl.BlockSpec((1, DH), lambda i: (0, 0)),
        ],
        out_specs=[
            pl.BlockSpec((RB, DH), lambda i: (i, 0)),
            pl.BlockSpec((2, RB, 128), lambda i: (0, i, 0)),
        ],
        out_shape=[
            jax.ShapeDtypeStruct((N, DH), F32),
            jax.ShapeDtypeStruct((2, N, 128), F32),
        ],
    )(drh, dinv16, xh, z, hl, whh, bh)


def _layer_weights(p):
    top = jnp.concatenate([p["Wxz"], p["Wxr"]], axis=1)
    bot = jnp.concatenate([p["Whz"], p["Whr"]], axis=1)
    wzr = jnp.concatenate([top, bot], axis=0).reshape(4, 128, 2 * DH)
    bzr = jnp.concatenate([p["bxz"] + p["bhz"],
                           p["bxr"] + p["bhr"]]).reshape(1, 2 * DH)
    wxh = p["Wxh"].reshape(2, 128, DH)
    whh = p["Whh"].reshape(2, 128, DH)
    bh = (p["bxh"] + p["bhh"]).reshape(1, DH)
    return wzr, bzr, wxh, whh, bh


def kernel(inp, edgidx, h, params):
    src = edgidx[0].astype(jnp.int32)
    dst = edgidx[1].astype(jnp.int32)
    e = src.shape[0]
    nblkd = 2 * (-(-e // (2 * NS * EB)))
    padd = NS * nblkd * EB - e
    dst3 = jnp.concatenate(
        [dst, jnp.full((padd,), DUMMY, jnp.int32)]).reshape(NS, nblkd, EB)
    nblkp = 2 * (-(-e // (2 * NS * EBP)))
    padp = NS * nblkp * EBP - e
    srcf = jnp.concatenate(
        [src, jnp.zeros((padp,), jnp.int32)]).reshape(NS, nblkp * EBP)
    dstp = jnp.concatenate(
        [dst, jnp.full((padp,), DUMMY, jnp.int32)]).reshape(NS, nblkp, EBP)

    prop6 = _make_prop(6, nblkp)
    prop2 = _make_prop(2, nblkp)

    ones = jnp.ones((EB, 128), F32)
    init = jnp.stack([jnp.ones((TR, 128), F32), jnp.zeros((TR, 128), F32)])
    degp1 = _make_deg(nblkd)(dst3, ones, init).reshape(2, N, 128)

    h1, h2 = h[0], h[1]
    dinv16, y6 = _scale_call(degp1, inp, h1, h2)

    d6 = prop6(srcf, dstp, y6.reshape(6 * N, 128)).reshape(6, N, 128)
    dx1, dh1, dh2 = d6[0:2], d6[2:4], d6[4:6]

    wzr1, bzr1, wxh1, whh1, bh1 = _layer_weights(params[0])
    wzr2, bzr2, wxh2, whh2, bh2 = _layer_weights(params[1])

    z1, yrh1, xh1 = _zr_call(dx1, dh1, dinv16, h1, wzr1, bzr1, wxh1)
    drh1 = prop2(srcf, dstp, yrh1.reshape(2 * N, 128)).reshape(2, N, 128)
    hp1, yx2 = _h_call(drh1, dinv16, xh1, z1, h1, whh1, bh1)

    dx2 = prop2(srcf, dstp, yx2.reshape(2 * N, 128)).reshape(2, N, 128)
    z2, yrh2, xh2 = _zr_call(dx2, dh2, dinv16, h2, wzr2, bzr2, wxh2)
    drh2 = prop2(srcf, dstp, yrh2.reshape(2 * N, 128)).reshape(2, N, 128)
    hp2, _ = _h_call(drh2, dinv16, xh2, z2, h2, whh2, bh2)

    h_out = jnp.stack([hp1, hp2], axis=0)
    return (h_out, h_out)
